# baseline (speedup 1.0000x reference)
import jax
import jax.numpy as jnp
from jax import lax
from jax.experimental import pallas as pl
from jax.experimental.pallas import tpu as pltpu
from jax.experimental.pallas import tpu_sc as plsc

NU = 25000
NT = 16
HALF = 25088
NP = 2 * HALF
HP = NP // 2
D = 64
E = 800000
RPT = HALF // NT
EPC = 802816
PAD = EPC - E
ET = EPC // NT
K = 224
NCH = ET // K
CN = 224
PR = CN // 2
NRC = RPT // CN

_MESH = plsc.VectorSubcoreMesh(core_axis_name="c", subcore_axis_name="s")
_SC_PARAMS = pltpu.CompilerParams(use_tc_tiling_on_sc=False)


def _sc_deg_body(sd_hbm, ones_hbm, rep_hbm, dacc,
                 ones_v, dtile, repb,
                 i0, i1, i2, i3, s0, s1, s2, s3, w0, w1):
    cid = lax.axis_index("c")
    sid = lax.axis_index("s")
    ebase = cid * EPC + sid * ET
    rbase = sid * RPT
    ib = (i0, i1, i2, i3)
    sg = (s0, s1, s2, s3)
    sw = (w0, w1)
    pltpu.sync_copy(ones_hbm, ones_v)
    for q in range(RPT // K):
        pltpu.sync_copy(ones_v, dacc.at[pl.ds(rbase + q * K, K)])
    plsc.subcore_barrier()

    def idx_start(c, a):
        pltpu.make_async_copy(
            sd_hbm.at[pl.ds(ebase + c * K, K)], ib[a], sg[a]).start()

    def idx_wait(a):
        pltpu.make_async_copy(sd_hbm.at[pl.ds(0, K)], ib[a], sg[a]).wait()

    def scat(a, w):
        return pltpu.make_async_copy(ones_v, dacc.at[ib[a]], sw[w])

    idx_start(0, 0)
    idx_start(1, 1)

    @pl.loop(0, NCH, step=4)
    def _(j):
        for t in range(4):
            c = j + t
            a, w = t, t % 2
            idx_wait(a)

            @pl.when(c >= 2)
            def _():
                scat((t + 2) % 4, w).wait()

            pltpu.async_copy(ones_v, dacc.at[ib[a]], sw[w], add=True)

            @pl.when(c + 2 < NCH)
            def _():
                idx_start(c + 2, (t + 2) % 4)

    scat(2, 0).wait()
    scat(3, 1).wait()
    plsc.subcore_barrier()
    pbase = cid * (HALF // 2) + sid * (RPT // 2)
    for q in range(NRC):
        pltpu.sync_copy(dacc.at[pl.ds(rbase + q * CN, CN)], dtile)

        @pl.loop(0, PR)
        def _(i):
            r0 = dtile[2 * i, pl.ds(0, 16)]
            r1 = dtile[2 * i + 1, pl.ds(0, 16)]
            v0 = jnp.full((16,), r0[0], jnp.float32)
            v1 = jnp.full((16,), r1[0], jnp.float32)
            for u in range(4):
                repb[i, pl.ds(u * 16, 16)] = v0
            for u in range(4):
                repb[i, pl.ds(64 + u * 16, 16)] = v1

        pltpu.sync_copy(repb, rep_hbm.at[pl.ds(pbase + q * PR, PR)])


def _sc_deg(sd, ones16):
    f = pl.kernel(
        _sc_deg_body,
        out_type=jax.ShapeDtypeStruct((HP, 128), jnp.float32),
        mesh=_MESH,
        compiler_params=_SC_PARAMS,
        scratch_types=(
            [pltpu.VMEM_SHARED((HALF, 16), jnp.float32),
             pltpu.VMEM((K, 16), jnp.float32),
             pltpu.VMEM((CN, 16), jnp.float32),
             pltpu.VMEM((PR, 128), jnp.float32)]
            + [pltpu.VMEM((K,), jnp.int32)] * 4
            + [pltpu.SemaphoreType.DMA] * 6
        ),
    )
    return f(sd, ones16)


def _sc_agg_body(y_hbm, sd_hbm, z_hbm, acc,
                 gi0, gi1, gi2, gi3, si0, si1, si2, si3, rb0, rb1,
                 sg0, sg1, sg2, sg3, ss0, ss1, ss2, ss3,
                 sr0, sr1, sw0, sw1):
    cid = lax.axis_index("c")
    sid = lax.axis_index("s")
    sbase = cid * EPC + sid * ET
    gbase = (1 - cid) * EPC + sid * ET
    goff = (1 - cid) * HALF
    rbase = sid * RPT
    gi = (gi0, gi1, gi2, gi3)
    si = (si0, si1, si2, si3)
    rb = (rb0, rb1)
    sg = (sg0, sg1, sg2, sg3)
    ss = (ss0, ss1, ss2, ss3)
    sr = (sr0, sr1)
    sw = (sw0, sw1)
    pltpu.sync_copy(y_hbm.at[pl.ds(cid * HALF + rbase, RPT)],
                    acc.at[pl.ds(rbase, RPT)])
    plsc.subcore_barrier()

    def gat(a, r):
        return pltpu.make_async_copy(y_hbm.at[gi[a]], rb[r], sr[r])

    def scat(a, r):
        return pltpu.make_async_copy(rb[r], acc.at[si[a]], sw[r])

    goff_v = jnp.full((16,), goff, jnp.int32)

    def idx_start(c, a):
        pltpu.make_async_copy(
            sd_hbm.at[pl.ds(gbase + c * K, K)], gi[a], sg[a]).start()
        pltpu.make_async_copy(
            sd_hbm.at[pl.ds(sbase + c * K, K)], si[a], ss[a]).start()

    def gidx_ready(a):
        pltpu.make_async_copy(sd_hbm.at[pl.ds(0, K)], gi[a], sg[a]).wait()

        @pl.loop(0, K // 16)
        def _(v):
            gi[a][pl.ds(v * 16, 16)] = gi[a][pl.ds(v * 16, 16)] + goff_v

    def sidx_wait(a):
        pltpu.make_async_copy(sd_hbm.at[pl.ds(0, K)], si[a], ss[a]).wait()

    idx_start(0, 0)
    idx_start(1, 1)
    idx_start(2, 2)
    gidx_ready(0)
    gat(0, 0).start()

    @pl.loop(0, NCH, step=4)
    def _(j):
        for t in range(4):
            c = j + t
            a, r = t, t % 2
            an, rn = (t + 1) % 4, (t + 1) % 2
            gat(a, r).wait()

            @pl.when(c + 1 < NCH)
            def _():
                gidx_ready(an)

            @pl.when(c >= 1)
            def _():
                scat((t + 3) % 4, rn).wait()

            @pl.when(c + 1 < NCH)
            def _():
                gat(an, rn).start()

            sidx_wait(a)
            scat(a, r).start(add=True)

            @pl.when(c + 3 < NCH)
            def _():
                idx_start(c + 3, (t + 3) % 4)

    scat(3, 1).wait()
    plsc.subcore_barrier()
    pltpu.sync_copy(acc.at[pl.ds(rbase, RPT)],
                    z_hbm.at[pl.ds(cid * HALF + rbase, RPT)])


def _sc_agg(y, sd):
    f = pl.kernel(
        _sc_agg_body,
        out_type=jax.ShapeDtypeStruct((NP, D), jnp.float32),
        mesh=_MESH,
        compiler_params=_SC_PARAMS,
        scratch_types=(
            [pltpu.VMEM_SHARED((HALF, D), jnp.float32)]
            + [pltpu.VMEM((K,), jnp.int32)] * 8
            + [pltpu.VMEM((K, D), jnp.float32)] * 2
            + [pltpu.SemaphoreType.DMA] * 12
        ),
    )
    return f(y, sd)


_RP = 3136


def _tc_pre_body(x_ref, s_ref, w_ref, y_ref):
    di = lax.rsqrt(s_ref[...])
    y_ref[...] = jnp.dot(x_ref[...] * di, w_ref[...],
                         preferred_element_type=jnp.float32)


def _tc_mid_body(z_ref, s_ref, b_ref, w_ref, y_ref):
    di = lax.rsqrt(s_ref[...])
    a = jnp.maximum(z_ref[...] * di + b_ref[...], 0.0)
    y_ref[...] = jnp.dot(a * di, w_ref[...],
                         preferred_element_type=jnp.float32)


def _tc_post_body(z_ref, s_ref, b_ref, o_ref):
    o_ref[...] = jnp.maximum(
        z_ref[...] * lax.rsqrt(s_ref[...]) + b_ref[...], 0.0)


def _row_spec():
    return pl.BlockSpec((_RP, 128), lambda i: (i, 0))


def _full_spec(h, w):
    return pl.BlockSpec((h, w), lambda i: (0, 0))


def _tc_pre(xp, sp, Wb):
    return pl.pallas_call(
        _tc_pre_body,
        grid=(HP // _RP,),
        in_specs=[_row_spec(), _row_spec(), _full_spec(128, 128)],
        out_specs=_row_spec(),
        out_shape=jax.ShapeDtypeStruct((HP, 128), jnp.float32),
    )(xp, sp, Wb)


def _tc_mid(zp, sp, bp, Wb):
    return pl.pallas_call(
        _tc_mid_body,
        grid=(HP // _RP,),
        in_specs=[_row_spec(), _row_spec(), _full_spec(1, 128),
                  _full_spec(128, 128)],
        out_specs=_row_spec(),
        out_shape=jax.ShapeDtypeStruct((HP, 128), jnp.float32),
    )(zp, sp, bp, Wb)


def _tc_post(zp, sp, bp):
    return pl.pallas_call(
        _tc_post_body,
        grid=(HP // _RP,),
        in_specs=[_row_spec(), _row_spec(), _full_spec(1, 128)],
        out_specs=_row_spec(),
        out_shape=jax.ShapeDtypeStruct((HP, 128), jnp.float32),
    )(zp, sp, bp)


def _blockdiag(W):
    Z = jnp.zeros((D, D), jnp.float32)
    return jnp.concatenate([
        jnp.concatenate([W, Z], axis=1),
        jnp.concatenate([Z, W], axis=1),
    ], axis=0)


def kernel(edge_index, users_emb, items_emb, W1, b1, W2, b2):
    src = edge_index[0].astype(jnp.int32)
    dst = edge_index[1].astype(jnp.int32)
    padv = jnp.full((PAD,), NU, jnp.int32)
    sd = jnp.concatenate([src, padv, dst, padv])
    zpad = jnp.zeros((HALF - NU, D), jnp.float32)
    xp = jnp.concatenate(
        [users_emb, zpad, items_emb, zpad], axis=0).reshape(HP, 128)
    ones16 = jnp.ones((K, 16), jnp.float32)
    Wb1 = _blockdiag(W1)
    Wb2 = _blockdiag(W2)
    bp1 = jnp.concatenate([b1, b1]).reshape(1, 128)
    bp2 = jnp.concatenate([b2, b2]).reshape(1, 128)

    sp = _sc_deg(sd, ones16)
    y1p = _tc_pre(xp, sp, Wb1)
    z1 = _sc_agg(y1p.reshape(NP, D), sd)
    y2p = _tc_mid(z1.reshape(HP, 128), sp, bp1, Wb2)
    z2 = _sc_agg(y2p.reshape(NP, D), sd)
    x2 = _tc_post(z2.reshape(HP, 128), sp, bp2).reshape(NP, D)

    return (x2[:NU], users_emb, x2[HALF:HALF + NU], items_emb)


# speedup vs baseline: 1.0193x; 1.0193x over previous
import jax
import jax.numpy as jnp
from jax import lax
from jax.experimental import pallas as pl
from jax.experimental.pallas import tpu as pltpu
from jax.experimental.pallas import tpu_sc as plsc

NU = 25000
NT = 16
HALF = 25088
NP = 2 * HALF
D = 64
E = 800000
RPT = HALF // NT
EPC = 819200
PAD = EPC - E
ET = EPC // NT
K = 200
NCH = ET // K

_MESH = plsc.VectorSubcoreMesh(core_axis_name="c", subcore_axis_name="s")
_SC_PARAMS = pltpu.CompilerParams(use_tc_tiling_on_sc=False)


def _idx_copy(idx_hbm, off, ref, sem):
    return pltpu.make_async_copy(idx_hbm.at[pl.ds(off, K)], ref, sem)


def _sc_deg_body(sidx_hbm, ones_hbm, deg_hbm, dacc, ones_v,
                 si0, si1, si2, si3, ss0, ss1, ss2, ss3, sw0, sw1):
    cid = lax.axis_index("c")
    sid = lax.axis_index("s")
    ebase = cid * EPC + sid * ET
    rbase = sid * RPT
    si = (si0, si1, si2, si3)
    ss = (ss0, ss1, ss2, ss3)
    sw = (sw0, sw1)
    pltpu.sync_copy(ones_hbm.at[pl.ds(0, K)], ones_v)
    pltpu.sync_copy(ones_hbm.at[pl.ds(rbase, RPT)], dacc.at[pl.ds(rbase, RPT)])
    plsc.subcore_barrier()

    def scat(a, w):
        return pltpu.make_async_copy(ones_v, dacc.at[si[a]], sw[w])

    _idx_copy(sidx_hbm, ebase, si0, ss0).start()
    _idx_copy(sidx_hbm, ebase + K, si1, ss1).start()

    @pl.loop(0, NCH, step=4)
    def _(j):
        for t in range(4):
            c = j + t
            a, w = t, t % 2
            _idx_copy(sidx_hbm, ebase, si[a], ss[a]).wait()

            @pl.when(c >= 2)
            def _():
                scat((t + 2) % 4, w).wait()

            pltpu.async_copy(ones_v, dacc.at[si[a]], sw[w], add=True)

            @pl.when(c + 2 < NCH)
            def _():
                _idx_copy(sidx_hbm, ebase + (c + 2) * K,
                          si[(t + 2) % 4], ss[(t + 2) % 4]).start()

    scat(2, 0).wait()
    scat(3, 1).wait()
    plsc.subcore_barrier()
    pltpu.sync_copy(dacc.at[pl.ds(rbase, RPT)],
                    deg_hbm.at[pl.ds(cid * HALF + rbase, RPT)])


def _sc_deg(sidx, ones16):
    f = pl.kernel(
        _sc_deg_body,
        out_type=jax.ShapeDtypeStruct((NP, 16), jnp.float32),
        mesh=_MESH,
        compiler_params=_SC_PARAMS,
        scratch_types=[
            pltpu.VMEM_SHARED((HALF, 16), jnp.float32),
            pltpu.VMEM((K, 16), jnp.float32),
            pltpu.VMEM((K,), jnp.int32),
            pltpu.VMEM((K,), jnp.int32),
            pltpu.VMEM((K,), jnp.int32),
            pltpu.VMEM((K,), jnp.int32),
            pltpu.SemaphoreType.DMA,
            pltpu.SemaphoreType.DMA,
            pltpu.SemaphoreType.DMA,
            pltpu.SemaphoreType.DMA,
            pltpu.SemaphoreType.DMA,
            pltpu.SemaphoreType.DMA,
        ],
    )
    return f(sidx, ones16)


def _sc_agg_body(y_hbm, gidx_hbm, sidx_hbm, z_hbm, acc,
                 gi0, gi1, gi2, gi3, si0, si1, si2, si3, rb0, rb1,
                 sg0, sg1, sg2, sg3, ss0, ss1, ss2, ss3,
                 sr0, sr1, sw0, sw1):
    cid = lax.axis_index("c")
    sid = lax.axis_index("s")
    ebase = cid * EPC + sid * ET
    rbase = sid * RPT
    gi = (gi0, gi1, gi2, gi3)
    si = (si0, si1, si2, si3)
    rb = (rb0, rb1)
    sg = (sg0, sg1, sg2, sg3)
    ss = (ss0, ss1, ss2, ss3)
    sr = (sr0, sr1)
    sw = (sw0, sw1)
    pltpu.sync_copy(y_hbm.at[pl.ds(cid * HALF + rbase, RPT)],
                    acc.at[pl.ds(rbase, RPT)])
    plsc.subcore_barrier()

    def gat(a, r):
        return pltpu.make_async_copy(y_hbm.at[gi[a]], rb[r], sr[r])

    def scat(a, r):
        return pltpu.make_async_copy(rb[r], acc.at[si[a]], sw[r])

    def idx_start(c, a):
        _idx_copy(gidx_hbm, ebase + c * K, gi[a], sg[a]).start()
        _idx_copy(sidx_hbm, ebase + c * K, si[a], ss[a]).start()

    idx_start(0, 0)
    idx_start(1, 1)
    idx_start(2, 2)
    _idx_copy(gidx_hbm, ebase, gi0, sg0).wait()
    gat(0, 0).start()

    @pl.loop(0, NCH, step=4)
    def _(j):
        for t in range(4):
            c = j + t
            a, r = t, t % 2
            an, rn = (t + 1) % 4, (t + 1) % 2
            gat(a, r).wait()

            @pl.when(c + 1 < NCH)
            def _():
                _idx_copy(gidx_hbm, ebase, gi[an], sg[an]).wait()

            @pl.when(c >= 1)
            def _():
                scat((t + 3) % 4, rn).wait()

            @pl.when(c + 1 < NCH)
            def _():
                gat(an, rn).start()

            _idx_copy(sidx_hbm, ebase, si[a], ss[a]).wait()
            scat(a, r).start(add=True)

            @pl.when(c + 3 < NCH)
            def _():
                idx_start(c + 3, (t + 3) % 4)

    scat(3, 1).wait()
    plsc.subcore_barrier()
    pltpu.sync_copy(acc.at[pl.ds(rbase, RPT)],
                    z_hbm.at[pl.ds(cid * HALF + rbase, RPT)])


def _sc_agg(y, gidx, sidx):
    f = pl.kernel(
        _sc_agg_body,
        out_type=jax.ShapeDtypeStruct((NP, D), jnp.float32),
        mesh=_MESH,
        compiler_params=_SC_PARAMS,
        scratch_types=(
            [pltpu.VMEM_SHARED((HALF, D), jnp.float32)]
            + [pltpu.VMEM((K,), jnp.int32)] * 8
            + [pltpu.VMEM((K, D), jnp.float32)] * 2
            + [pltpu.SemaphoreType.DMA] * 12
        ),
    )
    return f(y, gidx, sidx)


_R = 6272


def _dinv(deg_blk):
    return lax.rsqrt(deg_blk[:, 0:1])


def _tc_pre_body(x_ref, w_ref, deg_ref, y_ref):
    y_ref[...] = jnp.dot(x_ref[...], w_ref[...],
                         preferred_element_type=jnp.float32) * _dinv(deg_ref[...])


def _tc_mid_body(z_ref, deg_ref, b_ref, w_ref, y_ref):
    di = _dinv(deg_ref[...])
    a = jnp.maximum(z_ref[...] * di + b_ref[...], 0.0)
    y_ref[...] = jnp.dot(a, w_ref[...], preferred_element_type=jnp.float32) * di


def _tc_post_body(z_ref, deg_ref, b_ref, o_ref):
    o_ref[...] = jnp.maximum(
        z_ref[...] * _dinv(deg_ref[...]) + b_ref[...], 0.0)


def _row_spec(w):
    return pl.BlockSpec((_R, w), lambda i: (i, 0))


def _full_spec(h, w):
    return pl.BlockSpec((h, w), lambda i: (0, 0))


def _tc_pre(x, W, deg):
    return pl.pallas_call(
        _tc_pre_body,
        grid=(NP // _R,),
        in_specs=[_row_spec(D), _full_spec(D, D), _row_spec(16)],
        out_specs=_row_spec(D),
        out_shape=jax.ShapeDtypeStruct((NP, D), jnp.float32),
    )(x, W, deg)


def _tc_mid(z, deg, b, W):
    return pl.pallas_call(
        _tc_mid_body,
        grid=(NP // _R,),
        in_specs=[_row_spec(D), _row_spec(16), _full_spec(1, D),
                  _full_spec(D, D)],
        out_specs=_row_spec(D),
        out_shape=jax.ShapeDtypeStruct((NP, D), jnp.float32),
    )(z, deg, b, W)


def _tc_post(z, deg, b):
    return pl.pallas_call(
        _tc_post_body,
        grid=(NP // _R,),
        in_specs=[_row_spec(D), _row_spec(16), _full_spec(1, D)],
        out_specs=_row_spec(D),
        out_shape=jax.ShapeDtypeStruct((NP, D), jnp.float32),
    )(z, deg, b)


def kernel(edge_index, users_emb, items_emb, W1, b1, W2, b2):
    src = edge_index[0].astype(jnp.int32)
    dst = edge_index[1].astype(jnp.int32)
    pg = (jnp.arange(PAD, dtype=jnp.int32) * 131) % NU
    ps = NU + (jnp.arange(PAD, dtype=jnp.int32) % (HALF - NU))
    gidx = jnp.concatenate([dst + HALF, pg + HALF, src, pg])
    sidx = jnp.concatenate([src, ps, dst, ps])
    pad = jnp.zeros((HALF - NU, D), jnp.float32)
    x = jnp.concatenate([users_emb, pad, items_emb, pad], axis=0)
    ones16 = jnp.ones((HALF, 16), jnp.float32)

    deg = _sc_deg(sidx, ones16)
    y1 = _tc_pre(x, W1, deg)
    z1 = _sc_agg(y1, gidx, sidx)
    y2 = _tc_mid(z1, deg, b1.reshape(1, D), W2)
    z2 = _sc_agg(y2, gidx, sidx)
    x2 = _tc_post(z2, deg, b2.reshape(1, D))

    return (x2[:NU], users_emb, x2[HALF:HALF + NU], items_emb)
